# Initial kernel scaffold; baseline (speedup 1.0000x reference)
import jax
import jax.numpy as jnp
from jax.experimental import pallas as pl


def kernel(x, w1, b1, w2, b2, wf1, bf1, wf2, bf2, wf3, bf3):
    raise NotImplementedError("write your pallas kernel here")


# speedup vs baseline: 6.3181x; 6.3181x over previous
import jax
import jax.numpy as jnp
from jax.experimental import pallas as pl
from jax.experimental.pallas import tpu as pltpu

K = 5
S1 = 32
S2 = 64
OUT1 = 864
ACC1 = 904
IN1 = 1040
ACC2 = 600
B = 16
LANES = 128
FC_TILE = 512


def _convs_kernel(x_ref, w1_ref, w2a_ref, w2b_ref, b1_ref, b2a_ref, b2b_ref,
                  o_ref, acc1, a1, acc2a, acc2b):
    for idx in range(K * K):
        i, j = idx // K, idx % K
        s = i * S1 + j
        p = jnp.dot(x_ref[s:s + ACC1, :], w1_ref[idx * LANES:(idx + 1) * LANES, :],
                    preferred_element_type=jnp.float32)
        if idx == 0:
            acc1[...] = p
        else:
            acc1[...] += p

    q = jnp.maximum(acc1[0:OUT1, :], acc1[1:OUT1 + 1, :])
    q = jnp.maximum(q, acc1[S1:S1 + OUT1, :])
    q = jnp.maximum(q, acc1[S1 + 1:S1 + 1 + OUT1, :])
    a1[...] = jnp.maximum(q + b1_ref[...], 0.0)

    for half, (w2_ref, acc2) in enumerate(((w2a_ref, acc2a), (w2b_ref, acc2b))):
        for idx in range(K * K):
            i, j = idx // K, idx % K
            s = i * S2 + 2 * j
            p = jnp.dot(a1[s:s + ACC2, :], w2_ref[idx * LANES:(idx + 1) * LANES, :],
                        preferred_element_type=jnp.float32)
            if idx == 0:
                acc2[...] = p
            else:
                acc2[...] += p

    for half, (acc2, b2_ref) in enumerate(((acc2a, b2a_ref), (acc2b, b2b_ref))):
        for idx in range(K * K):
            ph, pw = idx // K, idx % K
            base = ph * 2 * S2 + 4 * pw
            v = jnp.maximum(
                jnp.maximum(acc2[base:base + 1, :], acc2[base + 2:base + 3, :]),
                jnp.maximum(acc2[base + S2:base + S2 + 1, :],
                            acc2[base + S2 + 2:base + S2 + 3, :]))
            r = half * 32 + idx
            o_ref[r:r + 1, :] = jnp.maximum(v + b2_ref[...], 0.0)


def _fc_kernel(x_ref, wf1_ref, bf1_ref, wf2_ref, bf2_ref, wf3_ref, bf3_ref,
               o_ref):
    h1 = jnp.maximum(
        jnp.dot(x_ref[...], wf1_ref[...], preferred_element_type=jnp.float32)
        + bf1_ref[...], 0.0)
    h2 = jnp.maximum(
        jnp.dot(h1, wf2_ref[...], preferred_element_type=jnp.float32)
        + bf2_ref[...], 0.0)
    o_ref[...] = (jnp.dot(h2, wf3_ref[...], preferred_element_type=jnp.float32)
                  + bf3_ref[...])


def _block_diag(w):
    eye = jnp.eye(B, dtype=w.dtype)
    return jnp.einsum('ab,tij->taibj', eye, w).reshape(K * K * LANES, LANES)


@jax.jit
def kernel(x, w1, b1, w2, b2, wf1, bf1, wf2, bf2, wf3, bf3):
    n = x.shape[0]
    nb = n // B

    y = x.reshape(nb, B, 3, 1024).transpose(0, 3, 1, 2)
    y = jnp.pad(y, ((0, 0), (0, IN1 - 1024), (0, 0), (0, 5)))
    y = y.reshape(nb * IN1, LANES)

    w1bd = _block_diag(w1.reshape(K * K, 8, LANES)[:, :, :8])
    w2s = w2.reshape(K * K, LANES, LANES)[:, :8, :16]
    w2a = _block_diag(w2s[:, :, :8])
    w2b = _block_diag(w2s[:, :, 8:])
    b1p = jnp.tile(b1[:, :8], (1, B))
    b2a = jnp.tile(b2[:, :8], (1, B))
    b2b = jnp.tile(b2[:, 8:16], (1, B))

    feats = pl.pallas_call(
        _convs_kernel,
        out_shape=jax.ShapeDtypeStruct((nb * 64, LANES), jnp.float32),
        grid_spec=pltpu.PrefetchScalarGridSpec(
            num_scalar_prefetch=0,
            grid=(nb,),
            in_specs=[
                pl.BlockSpec((IN1, LANES), lambda b: (b, 0)),
                pl.BlockSpec((K * K * LANES, LANES), lambda b: (0, 0)),
                pl.BlockSpec((K * K * LANES, LANES), lambda b: (0, 0)),
                pl.BlockSpec((K * K * LANES, LANES), lambda b: (0, 0)),
                pl.BlockSpec((1, LANES), lambda b: (0, 0)),
                pl.BlockSpec((1, LANES), lambda b: (0, 0)),
                pl.BlockSpec((1, LANES), lambda b: (0, 0)),
            ],
            out_specs=pl.BlockSpec((64, LANES), lambda b: (b, 0)),
            scratch_shapes=[
                pltpu.VMEM((ACC1, LANES), jnp.float32),
                pltpu.VMEM((OUT1, LANES), jnp.float32),
                pltpu.VMEM((ACC2, LANES), jnp.float32),
                pltpu.VMEM((ACC2, LANES), jnp.float32),
            ],
        ),
        compiler_params=pltpu.CompilerParams(dimension_semantics=("parallel",)),
    )(y, w1bd, w2a, w2b, b1p, b2a, b2b)

    f = feats.reshape(nb, 2, 32, B, 8)[:, :, :25]
    X = f.transpose(0, 3, 1, 4, 2).reshape(n, 400)
    n_pad = pl.cdiv(n, FC_TILE) * FC_TILE
    X = jnp.pad(X, ((0, n_pad - n), (0, 112)))

    wf1r = wf1.reshape(K * K, LANES, LANES)[:, :16, :].transpose(1, 0, 2)
    wf1r = jnp.pad(wf1r.reshape(400, LANES), ((0, 112), (0, 0)))

    out = pl.pallas_call(
        _fc_kernel,
        out_shape=jax.ShapeDtypeStruct((n_pad, LANES), jnp.float32),
        grid_spec=pltpu.PrefetchScalarGridSpec(
            num_scalar_prefetch=0,
            grid=(n_pad // FC_TILE,),
            in_specs=[
                pl.BlockSpec((FC_TILE, 512), lambda b: (b, 0)),
                pl.BlockSpec((512, LANES), lambda b: (0, 0)),
                pl.BlockSpec((1, LANES), lambda b: (0, 0)),
                pl.BlockSpec((LANES, LANES), lambda b: (0, 0)),
                pl.BlockSpec((1, LANES), lambda b: (0, 0)),
                pl.BlockSpec((LANES, LANES), lambda b: (0, 0)),
                pl.BlockSpec((1, LANES), lambda b: (0, 0)),
            ],
            out_specs=pl.BlockSpec((FC_TILE, LANES), lambda b: (b, 0)),
        ),
        compiler_params=pltpu.CompilerParams(dimension_semantics=("parallel",)),
    )(X, wf1r, bf1, wf2, bf2, wf3, bf3)

    return out[:n, :10]


# speedup vs baseline: 10.8041x; 1.7100x over previous
import jax
import jax.numpy as jnp
from jax.experimental import pallas as pl
from jax.experimental.pallas import tpu as pltpu

K = 5
S1 = 32
OUT1 = 864
ACC1 = 904
IN1 = 1040
SC = 16
A1C = 232
ACC2 = 160
B = 16
LANES = 128
FC_TILE = 512


def _pairmax(v):
    m = v.shape[0] // 2
    return jnp.max(v.reshape(m, 2, LANES), axis=1)


def _convs_kernel(x_ref, w1_ref, w2a_ref, w2b_ref, b1_ref, b2a_ref, b2b_ref,
                  o_ref, acc1, a1c, acc2a, acc2b):
    for idx in range(K * K):
        i, j = idx // K, idx % K
        s = i * S1 + j
        p = jnp.dot(x_ref[s:s + ACC1, :], w1_ref[idx * LANES:(idx + 1) * LANES, :],
                    preferred_element_type=jnp.float32)
        if idx == 0:
            acc1[...] = p
        else:
            acc1[...] += p

    for ph in range(14):
        v = _pairmax(acc1[64 * ph:64 * ph + 64, :])
        v = jnp.maximum(v[0:SC, :], v[SC:2 * SC, :])
        a1c[SC * ph:SC * (ph + 1), :] = jnp.maximum(v + b1_ref[...], 0.0)

    for half, (w2_ref, acc2) in enumerate(((w2a_ref, acc2a), (w2b_ref, acc2b))):
        for idx in range(K * K):
            i, j = idx // K, idx % K
            s = i * SC + j
            p = jnp.dot(a1c[s:s + ACC2, :], w2_ref[idx * LANES:(idx + 1) * LANES, :],
                        preferred_element_type=jnp.float32)
            if idx == 0:
                acc2[...] = p
            else:
                acc2[...] += p

    for half, (acc2, b2_ref) in enumerate(((acc2a, b2a_ref), (acc2b, b2b_ref))):
        for ph in range(5):
            v = _pairmax(acc2[32 * ph:32 * ph + 32, :])
            v = jnp.maximum(v[0:8, :], v[8:16, :])
            r = half * 40 + ph * 8
            o_ref[r:r + 8, :] = jnp.maximum(v + b2_ref[...], 0.0)


def _fc_kernel(x_ref, wf1_ref, bf1_ref, wf2_ref, bf2_ref, wf3_ref, bf3_ref,
               o_ref):
    h1 = jnp.maximum(
        jnp.dot(x_ref[...], wf1_ref[...], preferred_element_type=jnp.float32)
        + bf1_ref[...], 0.0)
    h2 = jnp.maximum(
        jnp.dot(h1, wf2_ref[...], preferred_element_type=jnp.float32)
        + bf2_ref[...], 0.0)
    o_ref[...] = (jnp.dot(h2, wf3_ref[...], preferred_element_type=jnp.float32)
                  + bf3_ref[...])


def _block_diag(w):
    eye = jnp.eye(B, dtype=w.dtype)
    return jnp.einsum('ab,tij->taibj', eye, w).reshape(K * K * LANES, LANES)


@jax.jit
def kernel(x, w1, b1, w2, b2, wf1, bf1, wf2, bf2, wf3, bf3):
    n = x.shape[0]
    nb = n // B

    y = x.reshape(nb, B, 3, 1024).transpose(0, 3, 1, 2)
    y = jnp.pad(y, ((0, 0), (0, IN1 - 1024), (0, 0), (0, 5)))
    y = y.reshape(nb * IN1, LANES)

    w1bd = _block_diag(w1.reshape(K * K, 8, LANES)[:, :, :8])
    w2s = w2.reshape(K * K, LANES, LANES)[:, :8, :16]
    w2a = _block_diag(w2s[:, :, :8])
    w2b = _block_diag(w2s[:, :, 8:])
    b1p = jnp.tile(b1[:, :8], (1, B))
    b2a = jnp.tile(b2[:, :8], (1, B))
    b2b = jnp.tile(b2[:, 8:16], (1, B))

    feats = pl.pallas_call(
        _convs_kernel,
        out_shape=jax.ShapeDtypeStruct((nb * 80, LANES), jnp.float32),
        grid_spec=pltpu.PrefetchScalarGridSpec(
            num_scalar_prefetch=0,
            grid=(nb,),
            in_specs=[
                pl.BlockSpec((IN1, LANES), lambda b: (b, 0)),
                pl.BlockSpec((K * K * LANES, LANES), lambda b: (0, 0)),
                pl.BlockSpec((K * K * LANES, LANES), lambda b: (0, 0)),
                pl.BlockSpec((K * K * LANES, LANES), lambda b: (0, 0)),
                pl.BlockSpec((1, LANES), lambda b: (0, 0)),
                pl.BlockSpec((1, LANES), lambda b: (0, 0)),
                pl.BlockSpec((1, LANES), lambda b: (0, 0)),
            ],
            out_specs=pl.BlockSpec((80, LANES), lambda b: (b, 0)),
            scratch_shapes=[
                pltpu.VMEM((ACC1, LANES), jnp.float32),
                pltpu.VMEM((A1C, LANES), jnp.float32),
                pltpu.VMEM((ACC2, LANES), jnp.float32),
                pltpu.VMEM((ACC2, LANES), jnp.float32),
            ],
        ),
        compiler_params=pltpu.CompilerParams(dimension_semantics=("parallel",)),
    )(y, w1bd, w2a, w2b, b1p, b2a, b2b)

    f = feats.reshape(nb, 2, 5, 8, B, 8)[:, :, :, :5]
    X = f.transpose(0, 4, 1, 5, 2, 3).reshape(n, 400)
    n_pad = pl.cdiv(n, FC_TILE) * FC_TILE
    X = jnp.pad(X, ((0, n_pad - n), (0, 112)))

    wf1r = wf1.reshape(K * K, LANES, LANES)[:, :16, :].transpose(1, 0, 2)
    wf1r = jnp.pad(wf1r.reshape(400, LANES), ((0, 112), (0, 0)))

    out = pl.pallas_call(
        _fc_kernel,
        out_shape=jax.ShapeDtypeStruct((n_pad, LANES), jnp.float32),
        grid_spec=pltpu.PrefetchScalarGridSpec(
            num_scalar_prefetch=0,
            grid=(n_pad // FC_TILE,),
            in_specs=[
                pl.BlockSpec((FC_TILE, 512), lambda b: (b, 0)),
                pl.BlockSpec((512, LANES), lambda b: (0, 0)),
                pl.BlockSpec((1, LANES), lambda b: (0, 0)),
                pl.BlockSpec((LANES, LANES), lambda b: (0, 0)),
                pl.BlockSpec((1, LANES), lambda b: (0, 0)),
                pl.BlockSpec((LANES, LANES), lambda b: (0, 0)),
                pl.BlockSpec((1, LANES), lambda b: (0, 0)),
            ],
            out_specs=pl.BlockSpec((FC_TILE, LANES), lambda b: (b, 0)),
        ),
        compiler_params=pltpu.CompilerParams(dimension_semantics=("parallel",)),
    )(X, wf1r, bf1, wf2, bf2, wf3, bf3)

    return out[:n, :10]


# speedup vs baseline: 13.5900x; 1.2579x over previous
import jax
import jax.numpy as jnp
from jax.experimental import pallas as pl
from jax.experimental.pallas import tpu as pltpu

K = 5
S1 = 32
ACC1 = 904
IN1 = 1040
SC = 16
A1C = 232
ACC2 = 160
B = 16
CL = 48
LANES = 128
FC_TILE = 512


def _pairmax(v):
    m = v.shape[0] // 2
    return jnp.max(v.reshape(m, 2, LANES), axis=1)


def _convs_kernel(x_ref, w1_ref, w2a_ref, w2b_ref, b1_ref, b2a_ref, b2b_ref,
                  o_ref, xs, acc1, a1c, acc2a, acc2b):
    xs[0:1024, :] = jnp.transpose(x_ref[...], (1, 0)).astype(jnp.bfloat16)
    xs[1024:IN1, :] = jnp.zeros((IN1 - 1024, CL), jnp.bfloat16)

    for idx in range(K * K):
        i, j = idx // K, idx % K
        s = i * S1 + j
        p = jnp.dot(xs[s:s + ACC1, :], w1_ref[idx * CL:(idx + 1) * CL, :],
                    preferred_element_type=jnp.float32)
        if idx == 0:
            acc1[...] = p
        else:
            acc1[...] += p

    for ph in range(14):
        v = _pairmax(acc1[64 * ph:64 * ph + 64, :])
        v = jnp.maximum(v[0:SC, :], v[SC:2 * SC, :])
        a1c[SC * ph:SC * (ph + 1), :] = jnp.maximum(
            v + b1_ref[...], 0.0).astype(jnp.bfloat16)

    for half, (w2_ref, acc2) in enumerate(((w2a_ref, acc2a), (w2b_ref, acc2b))):
        for idx in range(K * K):
            i, j = idx // K, idx % K
            s = i * SC + j
            p = jnp.dot(a1c[s:s + ACC2, :], w2_ref[idx * LANES:(idx + 1) * LANES, :],
                        preferred_element_type=jnp.float32)
            if idx == 0:
                acc2[...] = p
            else:
                acc2[...] += p

    for half, (acc2, b2_ref) in enumerate(((acc2a, b2a_ref), (acc2b, b2b_ref))):
        for ph in range(5):
            v = _pairmax(acc2[32 * ph:32 * ph + 32, :])
            v = jnp.maximum(v[0:8, :], v[8:16, :])
            r = half * 40 + ph * 8
            o_ref[r:r + 8, :] = jnp.maximum(v + b2_ref[...], 0.0)


def _fc_kernel(x_ref, wf1_ref, bf1_ref, wf2_ref, bf2_ref, wf3_ref, bf3_ref,
               o_ref):
    h1 = jnp.maximum(
        jnp.dot(x_ref[...], wf1_ref[...], preferred_element_type=jnp.float32)
        + bf1_ref[...], 0.0)
    h2 = jnp.maximum(
        jnp.dot(h1, wf2_ref[...], preferred_element_type=jnp.float32)
        + bf2_ref[...], 0.0)
    o_ref[...] = (jnp.dot(h2, wf3_ref[...], preferred_element_type=jnp.float32)
                  + bf3_ref[...])


def _block_diag(w, rows):
    eye = jnp.eye(B, dtype=w.dtype)
    return jnp.einsum('ab,tij->taibj', eye, w).reshape(K * K * B * rows, LANES)


@jax.jit
def kernel(x, w1, b1, w2, b2, wf1, bf1, wf2, bf2, wf3, bf3):
    n = x.shape[0]
    nb = n // B

    y = x.reshape(nb * B * 3, 1024)

    w1bd = _block_diag(w1.reshape(K * K, 8, LANES)[:, :3, :8], 3).astype(jnp.bfloat16)
    w2s = w2.reshape(K * K, LANES, LANES)[:, :8, :16]
    w2a = _block_diag(w2s[:, :, :8], 8).astype(jnp.bfloat16)
    w2b = _block_diag(w2s[:, :, 8:], 8).astype(jnp.bfloat16)
    b1p = jnp.tile(b1[:, :8], (1, B))
    b2a = jnp.tile(b2[:, :8], (1, B))
    b2b = jnp.tile(b2[:, 8:16], (1, B))

    feats = pl.pallas_call(
        _convs_kernel,
        out_shape=jax.ShapeDtypeStruct((nb * 80, LANES), jnp.float32),
        grid_spec=pltpu.PrefetchScalarGridSpec(
            num_scalar_prefetch=0,
            grid=(nb,),
            in_specs=[
                pl.BlockSpec((B * 3, 1024), lambda b: (b, 0)),
                pl.BlockSpec((K * K * CL, LANES), lambda b: (0, 0)),
                pl.BlockSpec((K * K * LANES, LANES), lambda b: (0, 0)),
                pl.BlockSpec((K * K * LANES, LANES), lambda b: (0, 0)),
                pl.BlockSpec((1, LANES), lambda b: (0, 0)),
                pl.BlockSpec((1, LANES), lambda b: (0, 0)),
                pl.BlockSpec((1, LANES), lambda b: (0, 0)),
            ],
            out_specs=pl.BlockSpec((80, LANES), lambda b: (b, 0)),
            scratch_shapes=[
                pltpu.VMEM((IN1, CL), jnp.bfloat16),
                pltpu.VMEM((ACC1, LANES), jnp.float32),
                pltpu.VMEM((A1C, LANES), jnp.bfloat16),
                pltpu.VMEM((ACC2, LANES), jnp.float32),
                pltpu.VMEM((ACC2, LANES), jnp.float32),
            ],
        ),
        compiler_params=pltpu.CompilerParams(dimension_semantics=("parallel",)),
    )(y, w1bd, w2a, w2b, b1p, b2a, b2b)

    f = feats.reshape(nb, 2, 5, 8, B, 8)[:, :, :, :5]
    X = f.transpose(0, 4, 1, 5, 2, 3).reshape(n, 400)
    n_pad = pl.cdiv(n, FC_TILE) * FC_TILE
    X = jnp.pad(X, ((0, n_pad - n), (0, 112)))

    wf1r = wf1.reshape(K * K, LANES, LANES)[:, :16, :].transpose(1, 0, 2)
    wf1r = jnp.pad(wf1r.reshape(400, LANES), ((0, 112), (0, 0)))

    out = pl.pallas_call(
        _fc_kernel,
        out_shape=jax.ShapeDtypeStruct((n_pad, LANES), jnp.float32),
        grid_spec=pltpu.PrefetchScalarGridSpec(
            num_scalar_prefetch=0,
            grid=(n_pad // FC_TILE,),
            in_specs=[
                pl.BlockSpec((FC_TILE, 512), lambda b: (b, 0)),
                pl.BlockSpec((512, LANES), lambda b: (0, 0)),
                pl.BlockSpec((1, LANES), lambda b: (0, 0)),
                pl.BlockSpec((LANES, LANES), lambda b: (0, 0)),
                pl.BlockSpec((1, LANES), lambda b: (0, 0)),
                pl.BlockSpec((LANES, LANES), lambda b: (0, 0)),
                pl.BlockSpec((1, LANES), lambda b: (0, 0)),
            ],
            out_specs=pl.BlockSpec((FC_TILE, LANES), lambda b: (b, 0)),
        ),
        compiler_params=pltpu.CompilerParams(dimension_semantics=("parallel",)),
    )(X, wf1r, bf1, wf2, bf2, wf3, bf3)

    return out[:n, :10]


# speedup vs baseline: 14.4209x; 1.0611x over previous
import jax
import jax.numpy as jnp
from jax.experimental import pallas as pl
from jax.experimental.pallas import tpu as pltpu

K = 5
S1 = 32
ACC1 = 904
IN1 = 1040
SC = 16
A1C = 232
ACC2 = 160
B = 16
CL = 48
LANES = 128
FC_TILE = 512


def _convs_kernel(x_ref, w1_ref, w2a_ref, w2b_ref, b1_ref, b2a_ref, b2b_ref,
                  o_ref, xs5, acc1, a1c, a1cs, acc2a, acc2b):
    xt = jnp.transpose(x_ref[...], (1, 0)).astype(jnp.bfloat16)
    for j in range(K):
        xs5[j * IN1:j * IN1 + 1024 - j, :] = xt[j:1024, :]
        xs5[j * IN1 + 1024 - j:(j + 1) * IN1, :] = jnp.zeros((IN1 - 1024 + j, CL),
                                                             jnp.bfloat16)

    for idx in range(K * K):
        i, j = idx // K, idx % K
        s = j * IN1 + i * S1
        p = jnp.dot(xs5[s:s + ACC1, :], w1_ref[idx * CL:(idx + 1) * CL, :],
                    preferred_element_type=jnp.float32)
        if idx == 0:
            acc1[...] = p
        else:
            acc1[...] += p

    for ph in range(14):
        r = 64 * ph
        v = jnp.maximum(
            jnp.maximum(acc1[r:r + 32:2, :], acc1[r + 1:r + 32:2, :]),
            jnp.maximum(acc1[r + 32:r + 64:2, :], acc1[r + 33:r + 64:2, :]))
        a1c[SC * ph:SC * (ph + 1), :] = jnp.maximum(
            v + b1_ref[...], 0.0).astype(jnp.bfloat16)

    for j in range(1, K):
        a1cs[(j - 1) * 224:j * 224, :] = a1c[j:j + 224, :]

    for half, (w2_ref, acc2) in enumerate(((w2a_ref, acc2a), (w2b_ref, acc2b))):
        for idx in range(K * K):
            i, j = idx // K, idx % K
            lhs = (a1c[i * SC:i * SC + ACC2, :] if j == 0 else
                   a1cs[(j - 1) * 224 + i * SC:(j - 1) * 224 + i * SC + ACC2, :])
            p = jnp.dot(lhs, w2_ref[idx * LANES:(idx + 1) * LANES, :],
                        preferred_element_type=jnp.float32)
            if idx == 0:
                acc2[...] = p
            else:
                acc2[...] += p

    for half, (acc2, b2_ref) in enumerate(((acc2a, b2a_ref), (acc2b, b2b_ref))):
        for ph in range(5):
            r = 32 * ph
            v = jnp.maximum(
                jnp.maximum(acc2[r:r + 16:2, :], acc2[r + 1:r + 16:2, :]),
                jnp.maximum(acc2[r + 16:r + 32:2, :], acc2[r + 17:r + 32:2, :]))
            ro = half * 40 + ph * 8
            o_ref[ro:ro + 8, :] = jnp.maximum(v + b2_ref[...], 0.0)


def _fc_kernel(x_ref, wf1_ref, bf1_ref, wf2_ref, bf2_ref, wf3_ref, bf3_ref,
               o_ref):
    h1 = jnp.maximum(
        jnp.dot(x_ref[...], wf1_ref[...], preferred_element_type=jnp.float32)
        + bf1_ref[...], 0.0)
    h2 = jnp.maximum(
        jnp.dot(h1, wf2_ref[...], preferred_element_type=jnp.float32)
        + bf2_ref[...], 0.0)
    o_ref[...] = (jnp.dot(h2, wf3_ref[...], preferred_element_type=jnp.float32)
                  + bf3_ref[...])


def _block_diag(w, rows):
    eye = jnp.eye(B, dtype=w.dtype)
    return jnp.einsum('ab,tij->taibj', eye, w).reshape(K * K * B * rows, LANES)


@jax.jit
def kernel(x, w1, b1, w2, b2, wf1, bf1, wf2, bf2, wf3, bf3):
    n = x.shape[0]
    nb = n // B

    y = x.reshape(nb * B * 3, 1024)

    w1bd = _block_diag(w1.reshape(K * K, 8, LANES)[:, :3, :8], 3).astype(jnp.bfloat16)
    w2s = w2.reshape(K * K, LANES, LANES)[:, :8, :16]
    w2a = _block_diag(w2s[:, :, :8], 8).astype(jnp.bfloat16)
    w2b = _block_diag(w2s[:, :, 8:], 8).astype(jnp.bfloat16)
    b1p = jnp.tile(b1[:, :8], (1, B))
    b2a = jnp.tile(b2[:, :8], (1, B))
    b2b = jnp.tile(b2[:, 8:16], (1, B))

    feats = pl.pallas_call(
        _convs_kernel,
        out_shape=jax.ShapeDtypeStruct((nb * 80, LANES), jnp.float32),
        grid_spec=pltpu.PrefetchScalarGridSpec(
            num_scalar_prefetch=0,
            grid=(nb,),
            in_specs=[
                pl.BlockSpec((B * 3, 1024), lambda b: (b, 0)),
                pl.BlockSpec((K * K * CL, LANES), lambda b: (0, 0)),
                pl.BlockSpec((K * K * LANES, LANES), lambda b: (0, 0)),
                pl.BlockSpec((K * K * LANES, LANES), lambda b: (0, 0)),
                pl.BlockSpec((1, LANES), lambda b: (0, 0)),
                pl.BlockSpec((1, LANES), lambda b: (0, 0)),
                pl.BlockSpec((1, LANES), lambda b: (0, 0)),
            ],
            out_specs=pl.BlockSpec((80, LANES), lambda b: (b, 0)),
            scratch_shapes=[
                pltpu.VMEM((5 * IN1, CL), jnp.bfloat16),
                pltpu.VMEM((ACC1, LANES), jnp.float32),
                pltpu.VMEM((A1C, LANES), jnp.bfloat16),
                pltpu.VMEM((4 * 224, LANES), jnp.bfloat16),
                pltpu.VMEM((ACC2, LANES), jnp.float32),
                pltpu.VMEM((ACC2, LANES), jnp.float32),
            ],
        ),
        compiler_params=pltpu.CompilerParams(dimension_semantics=("parallel",)),
    )(y, w1bd, w2a, w2b, b1p, b2a, b2b)

    f = feats.reshape(nb, 2, 5, 8, B, 8)[:, :, :, :5]
    X = f.transpose(0, 4, 1, 5, 2, 3).reshape(n, 400)
    n_pad = pl.cdiv(n, FC_TILE) * FC_TILE
    X = jnp.pad(X, ((0, n_pad - n), (0, 112)))

    wf1r = wf1.reshape(K * K, LANES, LANES)[:, :16, :].transpose(1, 0, 2)
    wf1r = jnp.pad(wf1r.reshape(400, LANES), ((0, 112), (0, 0)))

    out = pl.pallas_call(
        _fc_kernel,
        out_shape=jax.ShapeDtypeStruct((n_pad, LANES), jnp.float32),
        grid_spec=pltpu.PrefetchScalarGridSpec(
            num_scalar_prefetch=0,
            grid=(n_pad // FC_TILE,),
            in_specs=[
                pl.BlockSpec((FC_TILE, 512), lambda b: (b, 0)),
                pl.BlockSpec((512, LANES), lambda b: (0, 0)),
                pl.BlockSpec((1, LANES), lambda b: (0, 0)),
                pl.BlockSpec((LANES, LANES), lambda b: (0, 0)),
                pl.BlockSpec((1, LANES), lambda b: (0, 0)),
                pl.BlockSpec((LANES, LANES), lambda b: (0, 0)),
                pl.BlockSpec((1, LANES), lambda b: (0, 0)),
            ],
            out_specs=pl.BlockSpec((FC_TILE, LANES), lambda b: (b, 0)),
        ),
        compiler_params=pltpu.CompilerParams(dimension_semantics=("parallel",)),
    )(X, wf1r, bf1, wf2, bf2, wf3, bf3)

    return out[:n, :10]


# speedup vs baseline: 16.2776x; 1.1287x over previous
import jax
import jax.numpy as jnp
from jax.experimental import pallas as pl
from jax.experimental.pallas import tpu as pltpu

K = 5
S1 = 32
ACC1 = 904
IN1 = 1040
SC = 16
P2 = 224
ACC2 = 160
B = 16
G = 4
CL = 48
LANES = 128
M1 = (G - 1) * IN1 + ACC1
M2 = (G - 1) * P2 + ACC2
A1R = G * P2 + SC
FC_TILE = 512


def _convs_kernel(x_ref, w1_ref, w2a_ref, w2b_ref, b1_ref, b2a_ref, b2b_ref,
                  o_ref, xs5, acc1, a1c, a1cs, acc2a, acc2b):
    for g in range(G):
        xt = jnp.transpose(x_ref[g * 48:(g + 1) * 48, :], (1, 0)).astype(jnp.bfloat16)
        for j in range(K):
            r0 = j * G * IN1 + g * IN1
            xs5[r0:r0 + 1024 - j, :] = xt[j:1024, :]
            xs5[r0 + 1024 - j:r0 + IN1, :] = jnp.zeros((IN1 - 1024 + j, CL),
                                                       jnp.bfloat16)

    for idx in range(K * K):
        i, j = idx // K, idx % K
        s = j * G * IN1 + i * S1
        p = jnp.dot(xs5[s:s + M1, :], w1_ref[idx * CL:(idx + 1) * CL, :],
                    preferred_element_type=jnp.float32)
        if idx == 0:
            acc1[...] = p
        else:
            acc1[...] += p

    for g in range(G):
        for ph in range(14):
            r = g * IN1 + 64 * ph
            v = jnp.maximum(
                jnp.maximum(acc1[r:r + 32:2, :], acc1[r + 1:r + 32:2, :]),
                jnp.maximum(acc1[r + 32:r + 64:2, :], acc1[r + 33:r + 64:2, :]))
            q = g * P2 + SC * ph
            a1c[q:q + SC, :] = jnp.maximum(v + b1_ref[...], 0.0).astype(jnp.bfloat16)

    for j in range(1, K):
        a1cs[(j - 1) * G * P2:j * G * P2, :] = a1c[j:j + G * P2, :]

    for half, (w2_ref, acc2) in enumerate(((w2a_ref, acc2a), (w2b_ref, acc2b))):
        for idx in range(K * K):
            i, j = idx // K, idx % K
            lhs = (a1c[i * SC:i * SC + M2, :] if j == 0 else
                   a1cs[(j - 1) * G * P2 + i * SC:(j - 1) * G * P2 + i * SC + M2, :])
            p = jnp.dot(lhs, w2_ref[idx * LANES:(idx + 1) * LANES, :],
                        preferred_element_type=jnp.float32)
            if idx == 0:
                acc2[...] = p
            else:
                acc2[...] += p

    for half, (acc2, b2_ref) in enumerate(((acc2a, b2a_ref), (acc2b, b2b_ref))):
        for g in range(G):
            for ph in range(5):
                r = g * P2 + 32 * ph
                v = jnp.maximum(
                    jnp.maximum(acc2[r:r + 16:2, :], acc2[r + 1:r + 16:2, :]),
                    jnp.maximum(acc2[r + 16:r + 32:2, :], acc2[r + 17:r + 32:2, :]))
                ro = g * 80 + half * 40 + ph * 8
                o_ref[ro:ro + 8, :] = jnp.maximum(v + b2_ref[...], 0.0)


def _fc_kernel(x_ref, wf1_ref, bf1_ref, wf2_ref, bf2_ref, wf3_ref, bf3_ref,
               o_ref):
    h1 = jnp.maximum(
        jnp.dot(x_ref[...], wf1_ref[...], preferred_element_type=jnp.float32)
        + bf1_ref[...], 0.0)
    h2 = jnp.maximum(
        jnp.dot(h1, wf2_ref[...], preferred_element_type=jnp.float32)
        + bf2_ref[...], 0.0)
    o_ref[...] = (jnp.dot(h2, wf3_ref[...], preferred_element_type=jnp.float32)
                  + bf3_ref[...])


def _block_diag(w, rows):
    eye = jnp.eye(B, dtype=w.dtype)
    return jnp.einsum('ab,tij->taibj', eye, w).reshape(K * K * B * rows, LANES)


@jax.jit
def kernel(x, w1, b1, w2, b2, wf1, bf1, wf2, bf2, wf3, bf3):
    n = x.shape[0]
    nbg = n // (B * G)

    y = x.reshape(nbg * G * B * 3, 1024)

    w1bd = _block_diag(w1.reshape(K * K, 8, LANES)[:, :3, :8], 3).astype(jnp.bfloat16)
    w2s = w2.reshape(K * K, LANES, LANES)[:, :8, :16]
    w2a = _block_diag(w2s[:, :, :8], 8).astype(jnp.bfloat16)
    w2b = _block_diag(w2s[:, :, 8:], 8).astype(jnp.bfloat16)
    b1p = jnp.tile(b1[:, :8], (1, B))
    b2a = jnp.tile(b2[:, :8], (1, B))
    b2b = jnp.tile(b2[:, 8:16], (1, B))

    feats = pl.pallas_call(
        _convs_kernel,
        out_shape=jax.ShapeDtypeStruct((nbg * G * 80, LANES), jnp.float32),
        grid_spec=pltpu.PrefetchScalarGridSpec(
            num_scalar_prefetch=0,
            grid=(nbg,),
            in_specs=[
                pl.BlockSpec((G * B * 3, 1024), lambda b: (b, 0)),
                pl.BlockSpec((K * K * CL, LANES), lambda b: (0, 0)),
                pl.BlockSpec((K * K * LANES, LANES), lambda b: (0, 0)),
                pl.BlockSpec((K * K * LANES, LANES), lambda b: (0, 0)),
                pl.BlockSpec((1, LANES), lambda b: (0, 0)),
                pl.BlockSpec((1, LANES), lambda b: (0, 0)),
                pl.BlockSpec((1, LANES), lambda b: (0, 0)),
            ],
            out_specs=pl.BlockSpec((G * 80, LANES), lambda b: (b, 0)),
            scratch_shapes=[
                pltpu.VMEM((5 * G * IN1, CL), jnp.bfloat16),
                pltpu.VMEM((M1, LANES), jnp.float32),
                pltpu.VMEM((A1R, LANES), jnp.bfloat16),
                pltpu.VMEM((4 * G * P2, LANES), jnp.bfloat16),
                pltpu.VMEM((M2, LANES), jnp.float32),
                pltpu.VMEM((M2, LANES), jnp.float32),
            ],
        ),
        compiler_params=pltpu.CompilerParams(dimension_semantics=("parallel",)),
    )(y, w1bd, w2a, w2b, b1p, b2a, b2b)

    f = feats.reshape(nbg, G, 2, 5, 8, B, 8)[:, :, :, :, :5]
    X = f.transpose(0, 1, 5, 2, 6, 3, 4).reshape(n, 400)
    n_pad = pl.cdiv(n, FC_TILE) * FC_TILE
    X = jnp.pad(X, ((0, n_pad - n), (0, 112)))

    wf1r = wf1.reshape(K * K, LANES, LANES)[:, :16, :].transpose(1, 0, 2)
    wf1r = jnp.pad(wf1r.reshape(400, LANES), ((0, 112), (0, 0)))

    out = pl.pallas_call(
        _fc_kernel,
        out_shape=jax.ShapeDtypeStruct((n_pad, LANES), jnp.float32),
        grid_spec=pltpu.PrefetchScalarGridSpec(
            num_scalar_prefetch=0,
            grid=(n_pad // FC_TILE,),
            in_specs=[
                pl.BlockSpec((FC_TILE, 512), lambda b: (b, 0)),
                pl.BlockSpec((512, LANES), lambda b: (0, 0)),
                pl.BlockSpec((1, LANES), lambda b: (0, 0)),
                pl.BlockSpec((LANES, LANES), lambda b: (0, 0)),
                pl.BlockSpec((1, LANES), lambda b: (0, 0)),
                pl.BlockSpec((LANES, LANES), lambda b: (0, 0)),
                pl.BlockSpec((1, LANES), lambda b: (0, 0)),
            ],
            out_specs=pl.BlockSpec((FC_TILE, LANES), lambda b: (b, 0)),
        ),
        compiler_params=pltpu.CompilerParams(dimension_semantics=("parallel",)),
    )(X, wf1r, bf1, wf2, bf2, wf3, bf3)

    return out[:n, :10]


# speedup vs baseline: 19.6958x; 1.2100x over previous
import jax
import jax.numpy as jnp
from jax.experimental import pallas as pl
from jax.experimental.pallas import tpu as pltpu

K = 5
S1 = 32
ACC1 = 904
IN1 = 1040
SC = 16
P2 = 224
ACC2 = 160
B = 16
G = 4
CL = 48
LANES = 128
M1 = (G - 1) * IN1 + ACC1
M2 = (G - 1) * P2 + ACC2
A1R = G * P2 + SC
FC_TILE = 512


def _convs_kernel(x_ref, w1p_ref, w14_ref, w2a_ref, w2b_ref, b1_ref, b2a_ref,
                  b2b_ref, o_ref, xsp, xs4, acc1, a1c, a1cs, acc2a, acc2b):
    for g in range(G):
        xt = jnp.transpose(x_ref[g * 48:(g + 1) * 48, :], (1, 0)).astype(jnp.bfloat16)
        for p in range(2):
            r0 = p * G * IN1 + g * IN1
            for half48, j in ((slice(0, CL), 2 * p), (slice(CL, 2 * CL), 2 * p + 1)):
                xsp[r0:r0 + 1024 - j, half48] = xt[j:1024, :]
                xsp[r0 + 1024 - j:r0 + IN1, half48] = jnp.zeros(
                    (IN1 - 1024 + j, CL), jnp.bfloat16)
        r0 = g * IN1
        xs4[r0:r0 + 1020, :] = xt[4:1024, :]
        xs4[r0 + 1020:r0 + IN1, :] = jnp.zeros((IN1 - 1020, CL), jnp.bfloat16)

    first = True
    for i in range(K):
        for p in range(2):
            s = p * G * IN1 + i * S1
            q = jnp.dot(xsp[s:s + M1, :], w1p_ref[(i * 2 + p) * 96:(i * 2 + p + 1) * 96, :],
                        preferred_element_type=jnp.float32)
            if first:
                acc1[...] = q
                first = False
            else:
                acc1[...] += q
        s = i * S1
        acc1[...] += jnp.dot(xs4[s:s + M1, :], w14_ref[i * CL:(i + 1) * CL, :],
                             preferred_element_type=jnp.float32)

    for g in range(G):
        for ph in range(14):
            r = g * IN1 + 64 * ph
            v = jnp.maximum(
                jnp.maximum(acc1[r:r + 32:2, :], acc1[r + 1:r + 32:2, :]),
                jnp.maximum(acc1[r + 32:r + 64:2, :], acc1[r + 33:r + 64:2, :]))
            q = g * P2 + SC * ph
            a1c[q:q + SC, :] = jnp.maximum(v + b1_ref[...], 0.0).astype(jnp.bfloat16)

    for j in range(1, K):
        a1cs[(j - 1) * G * P2:j * G * P2, :] = a1c[j:j + G * P2, :]

    for half, (w2_ref, acc2) in enumerate(((w2a_ref, acc2a), (w2b_ref, acc2b))):
        for idx in range(K * K):
            i, j = idx // K, idx % K
            lhs = (a1c[i * SC:i * SC + M2, :] if j == 0 else
                   a1cs[(j - 1) * G * P2 + i * SC:(j - 1) * G * P2 + i * SC + M2, :])
            p = jnp.dot(lhs, w2_ref[idx * LANES:(idx + 1) * LANES, :],
                        preferred_element_type=jnp.float32)
            if idx == 0:
                acc2[...] = p
            else:
                acc2[...] += p

    for half, (acc2, b2_ref) in enumerate(((acc2a, b2a_ref), (acc2b, b2b_ref))):
        for g in range(G):
            for ph in range(5):
                r = g * P2 + 32 * ph
                v = jnp.maximum(
                    jnp.maximum(acc2[r:r + 16:2, :], acc2[r + 1:r + 16:2, :]),
                    jnp.maximum(acc2[r + 16:r + 32:2, :], acc2[r + 17:r + 32:2, :]))
                ro = g * 80 + half * 40 + ph * 8
                o_ref[ro:ro + 8, :] = jnp.maximum(
                    v + b2_ref[...], 0.0).astype(jnp.bfloat16)


def _fc_kernel(x_ref, wf1_ref, bf1_ref, wf2_ref, bf2_ref, wf3_ref, bf3_ref,
               o_ref):
    h1 = jnp.maximum(
        jnp.dot(x_ref[...], wf1_ref[...], preferred_element_type=jnp.float32)
        + bf1_ref[...], 0.0).astype(jnp.bfloat16)
    h2 = jnp.maximum(
        jnp.dot(h1, wf2_ref[...], preferred_element_type=jnp.float32)
        + bf2_ref[...], 0.0).astype(jnp.bfloat16)
    o_ref[...] = (jnp.dot(h2, wf3_ref[...], preferred_element_type=jnp.float32)
                  + bf3_ref[...])


def _block_diag(w, rows):
    eye = jnp.eye(B, dtype=w.dtype)
    return jnp.einsum('ab,tij->taibj', eye, w).reshape(K * K * B * rows, LANES)


@jax.jit
def kernel(x, w1, b1, w2, b2, wf1, bf1, wf2, bf2, wf3, bf3):
    n = x.shape[0]
    nbg = n // (B * G)

    y = x.reshape(nbg * G * B * 3, 1024)

    w1bd = _block_diag(w1.reshape(K * K, 8, LANES)[:, :3, :8], 3).astype(jnp.bfloat16)
    w1t = w1bd.reshape(K, K, CL, LANES)
    w1p = w1t[:, :4].reshape(K, 2, 2 * CL, LANES).reshape(10 * 96, LANES)
    w14 = w1t[:, 4].reshape(K * CL, LANES)
    w2s = w2.reshape(K * K, LANES, LANES)[:, :8, :16]
    w2a = _block_diag(w2s[:, :, :8], 8).astype(jnp.bfloat16)
    w2b = _block_diag(w2s[:, :, 8:], 8).astype(jnp.bfloat16)
    b1p = jnp.tile(b1[:, :8], (1, B))
    b2a = jnp.tile(b2[:, :8], (1, B))
    b2b = jnp.tile(b2[:, 8:16], (1, B))

    feats = pl.pallas_call(
        _convs_kernel,
        out_shape=jax.ShapeDtypeStruct((nbg * G * 80, LANES), jnp.bfloat16),
        grid_spec=pltpu.PrefetchScalarGridSpec(
            num_scalar_prefetch=0,
            grid=(nbg,),
            in_specs=[
                pl.BlockSpec((G * B * 3, 1024), lambda b: (b, 0)),
                pl.BlockSpec((10 * 96, LANES), lambda b: (0, 0)),
                pl.BlockSpec((K * CL, LANES), lambda b: (0, 0)),
                pl.BlockSpec((K * K * LANES, LANES), lambda b: (0, 0)),
                pl.BlockSpec((K * K * LANES, LANES), lambda b: (0, 0)),
                pl.BlockSpec((1, LANES), lambda b: (0, 0)),
                pl.BlockSpec((1, LANES), lambda b: (0, 0)),
                pl.BlockSpec((1, LANES), lambda b: (0, 0)),
            ],
            out_specs=pl.BlockSpec((G * 80, LANES), lambda b: (b, 0)),
            scratch_shapes=[
                pltpu.VMEM((2 * G * IN1, 2 * CL), jnp.bfloat16),
                pltpu.VMEM((G * IN1, CL), jnp.bfloat16),
                pltpu.VMEM((M1, LANES), jnp.float32),
                pltpu.VMEM((A1R, LANES), jnp.bfloat16),
                pltpu.VMEM((4 * G * P2, LANES), jnp.bfloat16),
                pltpu.VMEM((M2, LANES), jnp.float32),
                pltpu.VMEM((M2, LANES), jnp.float32),
            ],
        ),
        compiler_params=pltpu.CompilerParams(dimension_semantics=("parallel",)),
    )(y, w1p, w14, w2a, w2b, b1p, b2a, b2b)

    f = feats.reshape(nbg, G, 2, 5, 8, B, 8)[:, :, :, :, :5]
    X = f.transpose(0, 1, 5, 2, 6, 3, 4).reshape(n, 400)
    n_pad = pl.cdiv(n, FC_TILE) * FC_TILE
    X = jnp.pad(X, ((0, n_pad - n), (0, 112)))

    wf1r = wf1.reshape(K * K, LANES, LANES)[:, :16, :].transpose(1, 0, 2)
    wf1r = jnp.pad(wf1r.reshape(400, LANES), ((0, 112), (0, 0))).astype(jnp.bfloat16)
    wf2b = wf2.astype(jnp.bfloat16)
    wf3b = wf3.astype(jnp.bfloat16)

    out = pl.pallas_call(
        _fc_kernel,
        out_shape=jax.ShapeDtypeStruct((n_pad, LANES), jnp.float32),
        grid_spec=pltpu.PrefetchScalarGridSpec(
            num_scalar_prefetch=0,
            grid=(n_pad // FC_TILE,),
            in_specs=[
                pl.BlockSpec((FC_TILE, 512), lambda b: (b, 0)),
                pl.BlockSpec((512, LANES), lambda b: (0, 0)),
                pl.BlockSpec((1, LANES), lambda b: (0, 0)),
                pl.BlockSpec((LANES, LANES), lambda b: (0, 0)),
                pl.BlockSpec((1, LANES), lambda b: (0, 0)),
                pl.BlockSpec((LANES, LANES), lambda b: (0, 0)),
                pl.BlockSpec((1, LANES), lambda b: (0, 0)),
            ],
            out_specs=pl.BlockSpec((FC_TILE, LANES), lambda b: (b, 0)),
        ),
        compiler_params=pltpu.CompilerParams(dimension_semantics=("parallel",)),
    )(X, wf1r, bf1, wf2b, bf2, wf3b, bf3)

    return out[:n, :10]


# speedup vs baseline: 24.1991x; 1.2286x over previous
import functools

import jax
import jax.numpy as jnp
from jax.experimental import pallas as pl
from jax.experimental.pallas import tpu as pltpu

K = 5
S1 = 32
ACC1 = 904
IN1 = 1040
SC = 16
P2 = 224
ACC2 = 160
B = 16
G = 4
CL = 48
LANES = 128
M1 = (G - 1) * IN1 + ACC1
M2 = (G - 1) * P2 + ACC2
A1R = G * P2 + SC
FC_TILE = 512


def _convs_kernel(x_ref, w1p_ref, w14_ref, w2a_ref, w2b_ref, b1_ref, b2a_ref,
                  b2b_ref, o_ref, xsp, xs4, acc1, a1c, a1cs, acc2a, acc2b):
    for g in range(G):
        xt = jnp.transpose(x_ref[g * 48:(g + 1) * 48, :], (1, 0)).astype(jnp.bfloat16)
        for p in range(2):
            r0 = p * G * IN1 + g * IN1
            for half48, j in ((slice(0, CL), 2 * p), (slice(CL, 2 * CL), 2 * p + 1)):
                xsp[r0:r0 + 1024 - j, half48] = xt[j:1024, :]
                xsp[r0 + 1024 - j:r0 + IN1, half48] = jnp.zeros(
                    (IN1 - 1024 + j, CL), jnp.bfloat16)
        r0 = g * IN1
        xs4[r0:r0 + 1020, :] = xt[4:1024, :]
        xs4[r0 + 1020:r0 + IN1, :] = jnp.zeros((IN1 - 1020, CL), jnp.bfloat16)

    first = True
    for i in range(K):
        for p in range(2):
            s = p * G * IN1 + i * S1
            q = jnp.dot(xsp[s:s + M1, :], w1p_ref[(i * 2 + p) * 96:(i * 2 + p + 1) * 96, :],
                        preferred_element_type=jnp.float32)
            if first:
                acc1[...] = q
                first = False
            else:
                acc1[...] += q
        s = i * S1
        acc1[...] += jnp.dot(xs4[s:s + M1, :], w14_ref[i * CL:(i + 1) * CL, :],
                             preferred_element_type=jnp.float32)

    for g in range(G):
        for ph in range(14):
            r = g * IN1 + 64 * ph
            v = jnp.maximum(
                jnp.maximum(acc1[r:r + 32:2, :], acc1[r + 1:r + 32:2, :]),
                jnp.maximum(acc1[r + 32:r + 64:2, :], acc1[r + 33:r + 64:2, :]))
            q = g * P2 + SC * ph
            a1c[q:q + SC, :] = jnp.maximum(v + b1_ref[...], 0.0).astype(jnp.bfloat16)

    a1c[G * P2:A1R, :] = jnp.zeros((A1R - G * P2, LANES), jnp.bfloat16)

    for j in range(1, K):
        a1cs[(j - 1) * G * P2:j * G * P2, :] = a1c[j:j + G * P2, :]

    for half, (w2_ref, acc2) in enumerate(((w2a_ref, acc2a), (w2b_ref, acc2b))):
        for idx in range(K * K):
            i, j = idx // K, idx % K
            lhs = (a1c[i * SC:i * SC + M2, :] if j == 0 else
                   a1cs[(j - 1) * G * P2 + i * SC:(j - 1) * G * P2 + i * SC + M2, :])
            p = jnp.dot(lhs, w2_ref[idx * LANES:(idx + 1) * LANES, :],
                        preferred_element_type=jnp.float32)
            if idx == 0:
                acc2[...] = p
            else:
                acc2[...] += p

    for g in range(G):
        rows = []
        for acc2, b2_ref in ((acc2a, b2a_ref), (acc2b, b2b_ref)):
            for ph in range(5):
                r = g * P2 + 32 * ph
                v = jnp.maximum(
                    jnp.maximum(acc2[r:r + 16:2, :], acc2[r + 1:r + 16:2, :]),
                    jnp.maximum(acc2[r + 16:r + 32:2, :], acc2[r + 17:r + 32:2, :]))
                rows.append(jnp.maximum(v + b2_ref[...], 0.0))
        slab = jnp.concatenate(rows, axis=0)
        o_ref[g * LANES:(g + 1) * LANES, :] = jnp.transpose(slab, (1, 0))


def _fc_kernel(tile, f_ref, wf1_ref, bf1_ref, wf2_ref, bf2_ref, wf3_ref,
               bf3_ref, o_ref, xsc):
    for c in range(8):
        xsc[:, c * 80:(c + 1) * 80] = f_ref[c:8 * tile:8, :].astype(jnp.bfloat16)
    h1 = jnp.maximum(
        jnp.dot(xsc[...], wf1_ref[...], preferred_element_type=jnp.float32)
        + bf1_ref[...], 0.0).astype(jnp.bfloat16)
    h2 = jnp.maximum(
        jnp.dot(h1, wf2_ref[...], preferred_element_type=jnp.float32)
        + bf2_ref[...], 0.0).astype(jnp.bfloat16)
    o_ref[...] = (jnp.dot(h2, wf3_ref[...], preferred_element_type=jnp.float32)
                  + bf3_ref[...])


def _block_diag(w, rows):
    eye = jnp.eye(B, dtype=w.dtype)
    return jnp.einsum('ab,tij->taibj', eye, w).reshape(K * K * B * rows, LANES)


@jax.jit
def kernel(x, w1, b1, w2, b2, wf1, bf1, wf2, bf2, wf3, bf3):
    n = x.shape[0]
    nbg = n // (B * G)

    y = x.reshape(nbg * G * B * 3, 1024)

    w1bd = _block_diag(w1.reshape(K * K, 8, LANES)[:, :3, :8], 3).astype(jnp.bfloat16)
    w1t = w1bd.reshape(K, K, CL, LANES)
    w1p = w1t[:, :4].reshape(K, 2, 2 * CL, LANES).reshape(10 * 96, LANES)
    w14 = w1t[:, 4].reshape(K * CL, LANES)
    w2s = w2.reshape(K * K, LANES, LANES)[:, :8, :16]
    w2a = _block_diag(w2s[:, :, :8], 8).astype(jnp.bfloat16)
    w2b = _block_diag(w2s[:, :, 8:], 8).astype(jnp.bfloat16)
    b1p = jnp.tile(b1[:, :8], (1, B))
    b2a = jnp.tile(b2[:, :8], (1, B))
    b2b = jnp.tile(b2[:, 8:16], (1, B))

    feats = pl.pallas_call(
        _convs_kernel,
        out_shape=jax.ShapeDtypeStruct((nbg * G * LANES, 80), jnp.float32),
        grid_spec=pltpu.PrefetchScalarGridSpec(
            num_scalar_prefetch=0,
            grid=(nbg,),
            in_specs=[
                pl.BlockSpec((G * B * 3, 1024), lambda b: (b, 0)),
                pl.BlockSpec((10 * 96, LANES), lambda b: (0, 0)),
                pl.BlockSpec((K * CL, LANES), lambda b: (0, 0)),
                pl.BlockSpec((K * K * LANES, LANES), lambda b: (0, 0)),
                pl.BlockSpec((K * K * LANES, LANES), lambda b: (0, 0)),
                pl.BlockSpec((1, LANES), lambda b: (0, 0)),
                pl.BlockSpec((1, LANES), lambda b: (0, 0)),
                pl.BlockSpec((1, LANES), lambda b: (0, 0)),
            ],
            out_specs=pl.BlockSpec((G * LANES, 80), lambda b: (b, 0)),
            scratch_shapes=[
                pltpu.VMEM((2 * G * IN1, 2 * CL), jnp.bfloat16),
                pltpu.VMEM((G * IN1, CL), jnp.bfloat16),
                pltpu.VMEM((M1, LANES), jnp.float32),
                pltpu.VMEM((A1R, LANES), jnp.bfloat16),
                pltpu.VMEM((4 * G * P2, LANES), jnp.bfloat16),
                pltpu.VMEM((M2, LANES), jnp.float32),
                pltpu.VMEM((M2, LANES), jnp.float32),
            ],
        ),
        compiler_params=pltpu.CompilerParams(dimension_semantics=("parallel",)),
    )(y, w1p, w14, w2a, w2b, b1p, b2a, b2b)

    wf1r = wf1.reshape(K, K, LANES, LANES)[:, :, :16, :]
    wf1r = wf1r.reshape(K, K, 2, 8, LANES).transpose(3, 2, 0, 1, 4)
    wf1r = jnp.pad(wf1r, ((0, 0), (0, 0), (0, 0), (0, 3), (0, 0)))
    wf1r = wf1r.reshape(640, LANES).astype(jnp.bfloat16)
    wf2b = wf2.astype(jnp.bfloat16)
    wf3b = wf3.astype(jnp.bfloat16)

    tile = FC_TILE if n % FC_TILE == 0 else n
    out = pl.pallas_call(
        functools.partial(_fc_kernel, tile),
        out_shape=jax.ShapeDtypeStruct((n, LANES), jnp.float32),
        grid_spec=pltpu.PrefetchScalarGridSpec(
            num_scalar_prefetch=0,
            grid=(n // tile,),
            in_specs=[
                pl.BlockSpec((8 * tile, 80), lambda b: (b, 0)),
                pl.BlockSpec((640, LANES), lambda b: (0, 0)),
                pl.BlockSpec((1, LANES), lambda b: (0, 0)),
                pl.BlockSpec((LANES, LANES), lambda b: (0, 0)),
                pl.BlockSpec((1, LANES), lambda b: (0, 0)),
                pl.BlockSpec((LANES, LANES), lambda b: (0, 0)),
                pl.BlockSpec((1, LANES), lambda b: (0, 0)),
            ],
            out_specs=pl.BlockSpec((tile, LANES), lambda b: (b, 0)),
            scratch_shapes=[pltpu.VMEM((tile, 640), jnp.bfloat16)],
        ),
        compiler_params=pltpu.CompilerParams(dimension_semantics=("parallel",)),
    )(feats, wf1r, bf1, wf2b, bf2, wf3b, bf3)

    return out[:n, :10]
